# baseline (speedup 1.0000x reference)
import jax
import jax.numpy as jnp
from jax.experimental import pallas as pl
from jax.experimental.pallas import tpu as pltpu

TOPK = 2
TS = 256


def _combine_kernel(pooled_ref, w_ref, b_ref, lora_ref, base_ref, out_ref,
                    idx_ref, buf, sem):
    g = pl.program_id(0)
    total = pl.num_programs(0)
    B, E = pooled_ref.shape[0], w_ref.shape[0]
    S = lora_ref.shape[1]
    nt = S // TS

    @pl.when(g == 0)
    def _gate():
        gate = jax.lax.dot_general(
            pooled_ref[...], w_ref[...], (((1,), (1,)), ((), ())),
            preferred_element_type=jnp.float32,
            precision=jax.lax.Precision.HIGHEST,
        ) + b_ref[...]
        lane = jax.lax.broadcasted_iota(jnp.int32, (B, E), 1)
        rank = jnp.zeros((B, E), jnp.int32)
        for j in range(E):
            gj = gate[:, j:j + 1]
            beats = (gj > gate) | ((gj == gate) & (j < lane))
            rank = rank + beats.astype(jnp.int32)
        for bb in range(B):
            for k in range(TOPK):
                idx_ref[bb, k] = jnp.sum(
                    jnp.where(rank[bb:bb + 1, :] == k, lane[:1], 0))

    def start(gg, slot):
        b = gg // nt
        t = gg % nt
        for k in range(TOPK):
            e = idx_ref[b, k]
            pltpu.make_async_copy(
                lora_ref.at[b, pl.ds(t * TS, TS), e, :],
                buf.at[slot, k],
                sem.at[slot, k],
            ).start()

    p = jax.lax.rem(g, 2)

    @pl.when(g == 0)
    def _first():
        start(g, p)

    @pl.when(g + 1 < total)
    def _prefetch_next():
        start(g + 1, 1 - p)

    def wait(gg, slot):
        b = gg // nt
        t = gg % nt
        for k in range(TOPK):
            e = idx_ref[b, k]
            pltpu.make_async_copy(
                lora_ref.at[b, pl.ds(t * TS, TS), e, :],
                buf.at[slot, k],
                sem.at[slot, k],
            ).wait()

    wait(g, p)
    out_ref[0] = base_ref[0] + (buf[p, 0] + buf[p, 1])


def kernel(x, base_res, lora_results, W, b):
    B, S, D, E = lora_results.shape
    nt = S // TS

    pooled = x[:, -1, :]
    lora_t = lora_results.transpose(0, 1, 3, 2)
    b2 = b.reshape(1, E)

    out = pl.pallas_call(
        _combine_kernel,
        grid=(B * nt,),
        in_specs=[
            pl.BlockSpec((B, D), lambda g: (0, 0)),
            pl.BlockSpec((E, D), lambda g: (0, 0)),
            pl.BlockSpec((1, E), lambda g: (0, 0)),
            pl.BlockSpec(memory_space=pltpu.MemorySpace.HBM),
            pl.BlockSpec((1, TS, D), lambda g: (g // nt, g % nt, 0)),
        ],
        out_specs=pl.BlockSpec((1, TS, D), lambda g: (g // nt, g % nt, 0)),
        out_shape=jax.ShapeDtypeStruct((B, S, D), jnp.float32),
        scratch_shapes=[
            pltpu.SMEM((B, TOPK), jnp.int32),
            pltpu.VMEM((2, TOPK, TS, D), jnp.float32),
            pltpu.SemaphoreType.DMA((2, TOPK)),
        ],
    )(pooled, W, b2, lora_t, base_res)
    return out


# speedup vs baseline: 1.0933x; 1.0933x over previous
import jax
import jax.numpy as jnp
from jax.experimental import pallas as pl
from jax.experimental.pallas import tpu as pltpu

TOPK = 2
TS = 512
NSLOT = 4


def _combine_kernel(pooled_ref, w_ref, b_ref, lora_ref, base_ref, out_ref,
                    idx_ref, buf, sem):
    g = pl.program_id(0)
    total = pl.num_programs(0)
    B, E = pooled_ref.shape[0], w_ref.shape[0]
    S = lora_ref.shape[1]
    nt = S // TS

    @pl.when(g == 0)
    def _gate():
        gate = jax.lax.dot_general(
            pooled_ref[...], w_ref[...], (((1,), (1,)), ((), ())),
            preferred_element_type=jnp.float32,
            precision=jax.lax.Precision.HIGHEST,
        ) + b_ref[...]
        lane = jax.lax.broadcasted_iota(jnp.int32, (B, E), 1)
        rank = jnp.zeros((B, E), jnp.int32)
        for j in range(E):
            gj = gate[:, j:j + 1]
            beats = (gj > gate) | ((gj == gate) & (j < lane))
            rank = rank + beats.astype(jnp.int32)
        for bb in range(B):
            for k in range(TOPK):
                idx_ref[bb, k] = jnp.sum(
                    jnp.where(rank[bb:bb + 1, :] == k, lane[:1], 0))

    def start(gg, slot):
        b = gg // nt
        t = gg % nt
        for k in range(TOPK):
            e = idx_ref[b, k]
            pltpu.make_async_copy(
                lora_ref.at[b, pl.ds(t * TS, TS), e, :],
                buf.at[slot, k],
                sem.at[slot, k],
            ).start()

    p = jax.lax.rem(g, NSLOT)

    @pl.when(g == 0)
    def _first():
        start(g, p)
        start(g + 1, jax.lax.rem(g + 1, NSLOT))

    @pl.when(g + 2 < total)
    def _prefetch_next():
        start(g + 2, jax.lax.rem(g + 2, NSLOT))

    def wait(gg, slot):
        b = gg // nt
        t = gg % nt
        for k in range(TOPK):
            e = idx_ref[b, k]
            pltpu.make_async_copy(
                lora_ref.at[b, pl.ds(t * TS, TS), e, :],
                buf.at[slot, k],
                sem.at[slot, k],
            ).wait()

    wait(g, p)
    out_ref[0] = base_ref[0] + (buf[p, 0] + buf[p, 1])


def kernel(x, base_res, lora_results, W, b):
    B, S, D, E = lora_results.shape
    nt = S // TS

    pooled = x[:, -1, :]
    lora_t = lora_results.transpose(0, 1, 3, 2)
    b2 = b.reshape(1, E)

    out = pl.pallas_call(
        _combine_kernel,
        grid=(B * nt,),
        in_specs=[
            pl.BlockSpec((B, D), lambda g: (0, 0)),
            pl.BlockSpec((E, D), lambda g: (0, 0)),
            pl.BlockSpec((1, E), lambda g: (0, 0)),
            pl.BlockSpec(memory_space=pltpu.MemorySpace.HBM),
            pl.BlockSpec((1, TS, D), lambda g: (g // nt, g % nt, 0)),
        ],
        out_specs=pl.BlockSpec((1, TS, D), lambda g: (g // nt, g % nt, 0)),
        out_shape=jax.ShapeDtypeStruct((B, S, D), jnp.float32),
        scratch_shapes=[
            pltpu.SMEM((B, TOPK), jnp.int32),
            pltpu.VMEM((NSLOT, TOPK, TS, D), jnp.float32),
            pltpu.SemaphoreType.DMA((NSLOT, TOPK)),
        ],
    )(pooled, W, b2, lora_t, base_res)
    return out


# speedup vs baseline: 1.1016x; 1.0076x over previous
import jax
import jax.numpy as jnp
from jax.experimental import pallas as pl
from jax.experimental.pallas import tpu as pltpu

TOPK = 2
TS = 1024
NSLOT = 3


def _combine_kernel(pooled_ref, w_ref, b_ref, lora_ref, base_ref, out_ref,
                    idx_ref, buf, sem):
    g = pl.program_id(0)
    total = pl.num_programs(0)
    B, E = pooled_ref.shape[0], w_ref.shape[0]
    S = lora_ref.shape[1]
    nt = S // TS

    @pl.when(g == 0)
    def _gate():
        gate = jax.lax.dot_general(
            pooled_ref[...], w_ref[...], (((1,), (1,)), ((), ())),
            preferred_element_type=jnp.float32,
            precision=jax.lax.Precision.HIGHEST,
        ) + b_ref[...]
        lane = jax.lax.broadcasted_iota(jnp.int32, (B, E), 1)
        rank = jnp.zeros((B, E), jnp.int32)
        for j in range(E):
            gj = gate[:, j:j + 1]
            beats = (gj > gate) | ((gj == gate) & (j < lane))
            rank = rank + beats.astype(jnp.int32)
        for bb in range(B):
            for k in range(TOPK):
                idx_ref[bb, k] = jnp.sum(
                    jnp.where(rank[bb:bb + 1, :] == k, lane[:1], 0))

    def start(gg, slot):
        b = gg // nt
        t = gg % nt
        for k in range(TOPK):
            e = idx_ref[b, k]
            pltpu.make_async_copy(
                lora_ref.at[b, pl.ds(t * TS, TS), e, :],
                buf.at[slot, k],
                sem.at[slot, k],
            ).start()

    p = jax.lax.rem(g, NSLOT)

    @pl.when(g == 0)
    def _first():
        start(g, p)
        start(g + 1, jax.lax.rem(g + 1, NSLOT))

    @pl.when(g + 2 < total)
    def _prefetch_next():
        start(g + 2, jax.lax.rem(g + 2, NSLOT))

    def wait(gg, slot):
        b = gg // nt
        t = gg % nt
        for k in range(TOPK):
            e = idx_ref[b, k]
            pltpu.make_async_copy(
                lora_ref.at[b, pl.ds(t * TS, TS), e, :],
                buf.at[slot, k],
                sem.at[slot, k],
            ).wait()

    wait(g, p)
    out_ref[0] = base_ref[0] + (buf[p, 0] + buf[p, 1])


def kernel(x, base_res, lora_results, W, b):
    B, S, D, E = lora_results.shape
    nt = S // TS

    pooled = x[:, -1, :]
    lora_t = lora_results.transpose(0, 1, 3, 2)
    b2 = b.reshape(1, E)

    out = pl.pallas_call(
        _combine_kernel,
        grid=(B * nt,),
        in_specs=[
            pl.BlockSpec((B, D), lambda g: (0, 0)),
            pl.BlockSpec((E, D), lambda g: (0, 0)),
            pl.BlockSpec((1, E), lambda g: (0, 0)),
            pl.BlockSpec(memory_space=pltpu.MemorySpace.HBM),
            pl.BlockSpec((1, TS, D), lambda g: (g // nt, g % nt, 0)),
        ],
        out_specs=pl.BlockSpec((1, TS, D), lambda g: (g // nt, g % nt, 0)),
        out_shape=jax.ShapeDtypeStruct((B, S, D), jnp.float32),
        scratch_shapes=[
            pltpu.SMEM((B, TOPK), jnp.int32),
            pltpu.VMEM((NSLOT, TOPK, TS, D), jnp.float32),
            pltpu.SemaphoreType.DMA((NSLOT, TOPK)),
        ],
    )(pooled, W, b2, lora_t, base_res)
    return out
